# Initial kernel scaffold; baseline (speedup 1.0000x reference)
import jax
import jax.numpy as jnp
from jax.experimental import pallas as pl


def kernel(x, batch, W1, b1, W2, b2):
    raise NotImplementedError("write your pallas kernel here")


# speedup vs baseline: 3.7713x; 3.7713x over previous
import functools

import jax
import jax.numpy as jnp
from jax.experimental import pallas as pl

_G = 64


def _scores_kernel(x_ref, w1_ref, b1_ref, w2_ref, b2_ref, s_ref):
    x = x_ref[...].astype(jnp.bfloat16)
    w1 = w1_ref[...].T.astype(jnp.bfloat16)
    h = jnp.dot(x, w1, preferred_element_type=jnp.float32)
    h = jnp.maximum(h + b1_ref[0][None, :], 0.0).astype(jnp.bfloat16)
    s = jnp.dot(h, w2_ref[...].T.astype(jnp.bfloat16),
                preferred_element_type=jnp.float32)
    s_ref[...] = s[:, 0].reshape(s_ref.shape) + b2_ref[0, 0]


def _key_of(sch):
    bits = jax.lax.bitcast_convert_type(sch, jnp.int32)
    return jnp.where(bits < 0, (~bits) ^ jnp.int32(-2147483648), bits)


def _select_kernel(s_ref, b_ref, maskf_ref, loss_ref, *, nch, ch):
    gidx = jax.lax.broadcasted_iota(jnp.int32, (_G, ch), 0)
    cidx = jax.lax.broadcasted_iota(jnp.int32, (_G, ch), 1)

    def cnt(thr, strict):
        def body(i, acc):
            bch = b_ref[i, :]
            kch = _key_of(s_ref[i, :])
            onehot = bch[None, :] == gidx
            if strict:
                c = kch[None, :] > thr[:, None]
            else:
                c = kch[None, :] >= thr[:, None]
            m = jnp.logical_and(onehot, c)
            return acc + jnp.sum(m.astype(jnp.float32), axis=1)

        return jax.lax.fori_loop(0, nch, body, jnp.zeros((_G,), jnp.float32))

    counts = cnt(jnp.full((_G,), -2147483648, jnp.int32), False)
    k_per = jnp.maximum(1.0, jnp.floor(0.5 * counts))

    def bs_body(_, lohi):
        lo, hi = lohi
        mid = (lo >> 1) + (hi >> 1) + ((lo | hi) & 1)
        ge = cnt(mid, False) >= k_per
        lo = jnp.where(ge, mid, lo)
        hi = jnp.where(ge, hi, mid - 1)
        return lo, hi

    t_thr, _ = jax.lax.fori_loop(
        0, 32, bs_body,
        (jnp.full((_G,), -2147483648, jnp.int32),
         jnp.full((_G,), 2147483647, jnp.int32)),
    )
    r = k_per - cnt(t_thr, True)

    def cnt_eq_le(mid):
        def body(i, acc):
            bch = b_ref[i, :]
            kch = _key_of(s_ref[i, :])
            onehot = bch[None, :] == gidx
            m = (onehot & (kch[None, :] == t_thr[:, None])
                 & ((cidx + i * ch) <= mid[:, None]))
            return acc + jnp.sum(m.astype(jnp.float32), axis=1)

        return jax.lax.fori_loop(0, nch, body, jnp.zeros((_G,), jnp.float32))

    def bs2_body(_, lohi):
        lo, hi = lohi
        mid = lo + (hi - lo) // 2
        ge = cnt_eq_le(mid) >= r
        hi = jnp.where(ge, mid, hi)
        lo = jnp.where(ge, lo, mid + 1)
        return lo, hi

    i_thr, _ = jax.lax.fori_loop(
        0, 18, bs2_body,
        (jnp.zeros((_G,), jnp.int32),
         jnp.full((_G,), nch * ch - 1, jnp.int32)),
    )

    def final_body(i, carry):
        sel_cnt, sel_sum, tot_sum = carry
        bch = b_ref[i, :]
        sch = s_ref[i, :]
        kch = _key_of(sch)
        onehot = bch[None, :] == gidx
        gt = kch[None, :] > t_thr[:, None]
        eqsel = ((kch[None, :] == t_thr[:, None])
                 & ((cidx + i * ch) <= i_thr[:, None]))
        selg = (onehot & (gt | eqsel)).astype(jnp.float32)
        maskf_ref[i, :] = jnp.sum(selg, axis=0)
        onef = onehot.astype(jnp.float32)
        sel_cnt = sel_cnt + jnp.sum(selg, axis=1)
        sel_sum = sel_sum + jnp.sum(selg * sch[None, :], axis=1)
        tot_sum = tot_sum + jnp.sum(onef * sch[None, :], axis=1)
        return sel_cnt, sel_sum, tot_sum

    z = jnp.zeros((_G,), jnp.float32)
    sel_cnt, sel_sum, tot_sum = jax.lax.fori_loop(0, nch, final_body, (z, z, z))

    unsel_cnt = counts - sel_cnt
    unsel_sum = tot_sum - sel_sum
    sel_mean = sel_sum / jnp.maximum(sel_cnt, 1.0)
    unsel_mean = unsel_sum / jnp.maximum(unsel_cnt, 1.0)
    per_graph = jnp.where(
        unsel_cnt > 0.0,
        jnp.maximum(0.5 - (sel_mean - unsel_mean), 0.0),
        0.0,
    )
    bsz = (jnp.max(b_ref[...]) + 1).astype(jnp.float32)
    loss_ref[...] = (jnp.sum(per_graph) / bsz * 0.2).reshape(1, 1)


def _pool_kernel(x_ref, b_ref, m_ref, out_ref):
    @pl.when(pl.program_id(0) == 0)
    def _():
        out_ref[...] = jnp.zeros_like(out_ref)

    i = pl.program_id(0)
    bch = b_ref[i, :]
    mch = m_ref[i, :]
    tile = bch.shape[0]
    gidx = jax.lax.broadcasted_iota(jnp.int32, (_G, tile), 0)
    w = (bch[None, :] == gidx).astype(jnp.float32) * mch[None, :]
    out_ref[...] += jnp.dot(w, x_ref[...], preferred_element_type=jnp.float32, precision=jax.lax.Precision.HIGHEST)


def kernel(x, batch, W1, b1, W2, b2):
    n, f = x.shape
    ch = 5000 if n % 5000 == 0 else n
    nch = n // ch
    batch = batch.astype(jnp.int32)
    b1r = b1.reshape(1, -1)
    b2r = b2.reshape(1, 1)
    batch2 = batch.reshape(nch, ch)

    scores = pl.pallas_call(
        _scores_kernel,
        grid=(nch,),
        in_specs=[
            pl.BlockSpec((ch, f), lambda i: (i, 0)),
            pl.BlockSpec(W1.shape, lambda i: (0, 0)),
            pl.BlockSpec(b1r.shape, lambda i: (0, 0)),
            pl.BlockSpec(W2.shape, lambda i: (0, 0)),
            pl.BlockSpec(b2r.shape, lambda i: (0, 0)),
        ],
        out_specs=pl.BlockSpec((1, 1, ch), lambda i: (i, 0, 0)),
        out_shape=jax.ShapeDtypeStruct((nch, 1, ch), jnp.float32),
    )(x, W1, b1r, W2, b2r)
    scores = scores.reshape(nch, ch)

    maskf, loss = pl.pallas_call(
        functools.partial(_select_kernel, nch=nch, ch=ch),
        out_shape=[
            jax.ShapeDtypeStruct((nch, ch), jnp.float32),
            jax.ShapeDtypeStruct((1, 1), jnp.float32),
        ],
    )(scores, batch2)

    pooled = pl.pallas_call(
        _pool_kernel,
        grid=(nch,),
        in_specs=[
            pl.BlockSpec((ch, f), lambda i: (i, 0)),
            pl.BlockSpec((nch, ch), lambda i: (0, 0)),
            pl.BlockSpec((nch, ch), lambda i: (0, 0)),
        ],
        out_specs=pl.BlockSpec((_G, f), lambda i: (0, 0)),
        out_shape=jax.ShapeDtypeStruct((_G, f), jnp.float32),
    )(x, batch2, maskf)

    return pooled, loss[0, 0], maskf.reshape(n) > 0.5


# speedup vs baseline: 4.3072x; 1.1421x over previous
import functools

import jax
import jax.numpy as jnp
from jax.experimental import pallas as pl
from jax.experimental.pallas import tpu as pltpu

_G = 64


def _scores_kernel(x_ref, w1_ref, b1_ref, w2_ref, b2_ref, s_ref):
    x = x_ref[...].astype(jnp.bfloat16)
    w1 = w1_ref[...].T.astype(jnp.bfloat16)
    h = jnp.dot(x, w1, preferred_element_type=jnp.float32)
    h = jnp.maximum(h + b1_ref[0][None, :], 0.0).astype(jnp.bfloat16)
    s = jnp.dot(h, w2_ref[...].T.astype(jnp.bfloat16),
                preferred_element_type=jnp.float32)
    s_ref[...] = s[:, 0].reshape(s_ref.shape) + b2_ref[0, 0]


def _key_of(sch):
    bits = jax.lax.bitcast_convert_type(sch, jnp.int32)
    return jnp.where(bits < 0, (~bits) ^ jnp.int32(-2147483648), bits)


def _select_kernel(s_ref, b_ref, maskf_ref, loss_ref, km_ref, *, nch, ch):
    gidx = jax.lax.broadcasted_iota(jnp.int32, (_G, ch), 0)
    cidx = jax.lax.broadcasted_iota(jnp.int32, (_G, ch), 1)

    def pre_body(i, acc):
        bch = b_ref[i, :]
        kch = _key_of(s_ref[i, :])
        onehot = bch[None, :] == gidx
        km_ref[i] = jnp.where(onehot, kch[None, :], jnp.int32(-2147483648))
        return acc + jnp.sum(onehot.astype(jnp.float32), axis=1)

    counts = jax.lax.fori_loop(
        0, nch, pre_body, jnp.zeros((_G,), jnp.float32))

    def cnt(thr, strict):
        def body(i, acc):
            km = km_ref[i]
            if strict:
                m = km > thr[:, None]
            else:
                m = km >= thr[:, None]
            return acc + jnp.sum(m.astype(jnp.float32), axis=1)

        return jax.lax.fori_loop(0, nch, body, jnp.zeros((_G,), jnp.float32))

    k_per = jnp.maximum(1.0, jnp.floor(0.5 * counts))

    def bs_body(_, lohi):
        lo, hi = lohi
        mid = (lo >> 1) + (hi >> 1) + ((lo | hi) & 1)
        ge = cnt(mid, False) >= k_per
        lo = jnp.where(ge, mid, lo)
        hi = jnp.where(ge, hi, mid - 1)
        return lo, hi

    t_thr, _ = jax.lax.fori_loop(
        0, 32, bs_body,
        (jnp.full((_G,), -2147483648, jnp.int32),
         jnp.full((_G,), 2147483647, jnp.int32)),
    )
    r = k_per - cnt(t_thr, True)

    def cnt_eq_le(mid):
        def body(i, acc):
            bch = b_ref[i, :]
            kch = _key_of(s_ref[i, :])
            onehot = bch[None, :] == gidx
            m = (onehot & (kch[None, :] == t_thr[:, None])
                 & ((cidx + i * ch) <= mid[:, None]))
            return acc + jnp.sum(m.astype(jnp.float32), axis=1)

        return jax.lax.fori_loop(0, nch, body, jnp.zeros((_G,), jnp.float32))

    def bs2_body(_, lohi):
        lo, hi = lohi
        mid = lo + (hi - lo) // 2
        ge = cnt_eq_le(mid) >= r
        hi = jnp.where(ge, mid, hi)
        lo = jnp.where(ge, lo, mid + 1)
        return lo, hi

    i_thr, _ = jax.lax.fori_loop(
        0, 18, bs2_body,
        (jnp.zeros((_G,), jnp.int32),
         jnp.full((_G,), nch * ch - 1, jnp.int32)),
    )

    def final_body(i, carry):
        sel_cnt, sel_sum, tot_sum = carry
        bch = b_ref[i, :]
        sch = s_ref[i, :]
        kch = _key_of(sch)
        onehot = bch[None, :] == gidx
        gt = kch[None, :] > t_thr[:, None]
        eqsel = ((kch[None, :] == t_thr[:, None])
                 & ((cidx + i * ch) <= i_thr[:, None]))
        selg = (onehot & (gt | eqsel)).astype(jnp.float32)
        maskf_ref[i, :] = jnp.sum(selg, axis=0)
        onef = onehot.astype(jnp.float32)
        sel_cnt = sel_cnt + jnp.sum(selg, axis=1)
        sel_sum = sel_sum + jnp.sum(selg * sch[None, :], axis=1)
        tot_sum = tot_sum + jnp.sum(onef * sch[None, :], axis=1)
        return sel_cnt, sel_sum, tot_sum

    z = jnp.zeros((_G,), jnp.float32)
    sel_cnt, sel_sum, tot_sum = jax.lax.fori_loop(0, nch, final_body, (z, z, z))

    unsel_cnt = counts - sel_cnt
    unsel_sum = tot_sum - sel_sum
    sel_mean = sel_sum / jnp.maximum(sel_cnt, 1.0)
    unsel_mean = unsel_sum / jnp.maximum(unsel_cnt, 1.0)
    per_graph = jnp.where(
        unsel_cnt > 0.0,
        jnp.maximum(0.5 - (sel_mean - unsel_mean), 0.0),
        0.0,
    )
    bsz = (jnp.max(b_ref[...]) + 1).astype(jnp.float32)
    loss_ref[...] = (jnp.sum(per_graph) / bsz * 0.2).reshape(1, 1)


def _pool_kernel(x_ref, b_ref, m_ref, out_ref):
    @pl.when(pl.program_id(0) == 0)
    def _():
        out_ref[...] = jnp.zeros_like(out_ref)

    i = pl.program_id(0)
    bch = b_ref[i, :]
    mch = m_ref[i, :]
    tile = bch.shape[0]
    gidx = jax.lax.broadcasted_iota(jnp.int32, (_G, tile), 0)
    w = (bch[None, :] == gidx).astype(jnp.float32) * mch[None, :]
    out_ref[...] += jnp.dot(w, x_ref[...], preferred_element_type=jnp.float32, precision=jax.lax.Precision.HIGHEST)


def kernel(x, batch, W1, b1, W2, b2):
    n, f = x.shape
    ch = 5000 if n % 5000 == 0 else n
    nch = n // ch
    batch = batch.astype(jnp.int32)
    b1r = b1.reshape(1, -1)
    b2r = b2.reshape(1, 1)
    batch2 = batch.reshape(nch, ch)

    scores = pl.pallas_call(
        _scores_kernel,
        grid=(nch,),
        in_specs=[
            pl.BlockSpec((ch, f), lambda i: (i, 0)),
            pl.BlockSpec(W1.shape, lambda i: (0, 0)),
            pl.BlockSpec(b1r.shape, lambda i: (0, 0)),
            pl.BlockSpec(W2.shape, lambda i: (0, 0)),
            pl.BlockSpec(b2r.shape, lambda i: (0, 0)),
        ],
        out_specs=pl.BlockSpec((1, 1, ch), lambda i: (i, 0, 0)),
        out_shape=jax.ShapeDtypeStruct((nch, 1, ch), jnp.float32),
    )(x, W1, b1r, W2, b2r)
    scores = scores.reshape(nch, ch)

    maskf, loss = pl.pallas_call(
        functools.partial(_select_kernel, nch=nch, ch=ch),
        out_shape=[
            jax.ShapeDtypeStruct((nch, ch), jnp.float32),
            jax.ShapeDtypeStruct((1, 1), jnp.float32),
        ],
        scratch_shapes=[pltpu.VMEM((nch, _G, ch), jnp.int32)],
    )(scores, batch2)

    pooled = pl.pallas_call(
        _pool_kernel,
        grid=(nch,),
        in_specs=[
            pl.BlockSpec((ch, f), lambda i: (i, 0)),
            pl.BlockSpec((nch, ch), lambda i: (0, 0)),
            pl.BlockSpec((nch, ch), lambda i: (0, 0)),
        ],
        out_specs=pl.BlockSpec((_G, f), lambda i: (0, 0)),
        out_shape=jax.ShapeDtypeStruct((_G, f), jnp.float32),
    )(x, batch2, maskf)

    return pooled, loss[0, 0], maskf.reshape(n) > 0.5
